# baseline (speedup 1.0000x reference)
import functools

import jax
import jax.numpy as jnp
from jax import lax
from jax.experimental import pallas as pl
from jax.experimental.pallas import tpu as pltpu
from jax.experimental.pallas import tpu_sc as plsc

_N = 10000
_D = 128
_E = 320000

_L = 16
_NSUB = 16
_NCORE = 2
_NW = _NCORE * _NSUB
_B = 128
_CH = 82
_EPW = _B * _CH
_EPAD = _NW * _EPW
_NP = 10240
_NT = 10016
_RPT = _NP // _NSUB
_DR = _NP // _L
_DB = _DR // _B
_DRT = _DR // _NSUB

_ROWS = 1024


def _tc1_body(x_ref, w1_ref, a1s_ref, a1d_ref, wres_ref, bres_ref,
              h1_ref, as_ref, ad_ref, res_ref):
    xb = x_ref[...]
    h = jnp.dot(xb, w1_ref[...], preferred_element_type=jnp.float32)
    h1_ref[...] = h
    as_ref[...] = jnp.dot(h, a1s_ref[...], preferred_element_type=jnp.float32)
    ad_ref[...] = jnp.dot(h, a1d_ref[...], preferred_element_type=jnp.float32)
    res_ref[...] = (jnp.dot(xb, wres_ref[...], preferred_element_type=jnp.float32)
                    + bres_ref[...])


def _tc1(xp, W1, a1s, a1d, Wres, bres):
    g = _NP // _ROWS
    return pl.pallas_call(
        _tc1_body,
        grid=(g,),
        in_specs=[
            pl.BlockSpec((_ROWS, _D), lambda i: (i, 0)),
            pl.BlockSpec((_D, 32), lambda i: (0, 0)),
            pl.BlockSpec((32, 1), lambda i: (0, 0)),
            pl.BlockSpec((32, 1), lambda i: (0, 0)),
            pl.BlockSpec((_D, 64), lambda i: (0, 0)),
            pl.BlockSpec((1, 64), lambda i: (0, 0)),
        ],
        out_specs=[
            pl.BlockSpec((_ROWS, 32), lambda i: (i, 0)),
            pl.BlockSpec((_ROWS, 1), lambda i: (i, 0)),
            pl.BlockSpec((_ROWS, 1), lambda i: (i, 0)),
            pl.BlockSpec((_ROWS, 64), lambda i: (i, 0)),
        ],
        out_shape=[
            jax.ShapeDtypeStruct((_NP, 32), jnp.float32),
            jax.ShapeDtypeStruct((_NP, 1), jnp.float32),
            jax.ShapeDtypeStruct((_NP, 1), jnp.float32),
            jax.ShapeDtypeStruct((_NP, 64), jnp.float32),
        ],
    )(xp, W1, a1s, a1d, Wres, bres)


def _tc2_body(m0_ref, m1_ref, d0_ref, d1_ref, b1_ref, w2_ref, a2s_ref, a2d_ref,
              h2_ref, as_ref, ad_ref):
    num = m0_ref[...] + m1_ref[...]
    den = d0_ref[...] + d1_ref[...] + 1e-16
    z = jnp.maximum(num / den + b1_ref[...], 0.0)
    h2 = jnp.dot(z, w2_ref[...], preferred_element_type=jnp.float32)
    h2_ref[...] = h2
    as_ref[...] = jnp.dot(h2, a2s_ref[...], preferred_element_type=jnp.float32)
    ad_ref[...] = jnp.dot(h2, a2d_ref[...], preferred_element_type=jnp.float32)


def _tc2(m0, m1, d0, d1, b1, W2, a2s, a2d):
    g = _NP // _ROWS
    return pl.pallas_call(
        _tc2_body,
        grid=(g,),
        in_specs=[
            pl.BlockSpec((_ROWS, 32), lambda i: (i, 0)),
            pl.BlockSpec((_ROWS, 32), lambda i: (i, 0)),
            pl.BlockSpec((_ROWS, 1), lambda i: (i, 0)),
            pl.BlockSpec((_ROWS, 1), lambda i: (i, 0)),
            pl.BlockSpec((1, 32), lambda i: (0, 0)),
            pl.BlockSpec((32, 64), lambda i: (0, 0)),
            pl.BlockSpec((64, 1), lambda i: (0, 0)),
            pl.BlockSpec((64, 1), lambda i: (0, 0)),
        ],
        out_specs=[
            pl.BlockSpec((_ROWS, 64), lambda i: (i, 0)),
            pl.BlockSpec((_ROWS, 1), lambda i: (i, 0)),
            pl.BlockSpec((_ROWS, 1), lambda i: (i, 0)),
        ],
        out_shape=[
            jax.ShapeDtypeStruct((_NP, 64), jnp.float32),
            jax.ShapeDtypeStruct((_NP, 1), jnp.float32),
            jax.ShapeDtypeStruct((_NP, 1), jnp.float32),
        ],
    )(m0, m1, d0, d1, b1, W2, a2s, a2d)


def _tc3_body(m0_ref, m1_ref, d0_ref, d1_ref, b2_ref, res_ref, wfc_ref, bfc_ref,
              o_ref):
    num = m0_ref[...] + m1_ref[...]
    den = d0_ref[...] + d1_ref[...] + 1e-16
    z = jnp.maximum(num / den + b2_ref[...], 0.0)
    t = z + res_ref[...]
    o_ref[...] = jax.nn.sigmoid(
        jnp.dot(t, wfc_ref[...], preferred_element_type=jnp.float32) + bfc_ref[...])


def _tc3(m0, m1, d0, d1, b2, res, Wfc, bfc):
    g = _NP // _ROWS
    return pl.pallas_call(
        _tc3_body,
        grid=(g,),
        in_specs=[
            pl.BlockSpec((_ROWS, 64), lambda i: (i, 0)),
            pl.BlockSpec((_ROWS, 64), lambda i: (i, 0)),
            pl.BlockSpec((_ROWS, 1), lambda i: (i, 0)),
            pl.BlockSpec((_ROWS, 1), lambda i: (i, 0)),
            pl.BlockSpec((1, 64), lambda i: (0, 0)),
            pl.BlockSpec((_ROWS, 64), lambda i: (i, 0)),
            pl.BlockSpec((64, 1), lambda i: (0, 0)),
            pl.BlockSpec((1, 1), lambda i: (0, 0)),
        ],
        out_specs=pl.BlockSpec((_ROWS, 1), lambda i: (i, 0)),
        out_shape=jax.ShapeDtypeStruct((_NP, 1), jnp.float32),
    )(m0, m1, d0, d1, b2, res, Wfc, bfc)


def _make_sc_edge(F):
    mesh = plsc.VectorSubcoreMesh(core_axis_name="c", subcore_axis_name="s")

    @functools.partial(
        pl.kernel,
        out_type=[
            jax.ShapeDtypeStruct((_NCORE, _NP, F), jnp.float32),
            jax.ShapeDtypeStruct((_NCORE, _DR, _L), jnp.float32),
        ],
        mesh=mesh,
        compiler_params=pltpu.CompilerParams(needs_layout_passes=False,
                                             use_tc_tiling_on_sc=False),
        scratch_types=[
            pltpu.VMEM((_CH, _B), jnp.int32),
            pltpu.VMEM((_CH, _B), jnp.int32),
            pltpu.VMEM((_NT,), jnp.float32),
            pltpu.VMEM((_NT,), jnp.float32),
            pltpu.VMEM((_B, F), jnp.float32),
            pltpu.VMEM((_B, F), jnp.float32),
            pltpu.VMEM((_B, F), jnp.float32),
            pltpu.VMEM((_B, F), jnp.float32),
            pltpu.VMEM((_B,), jnp.float32),
            pltpu.VMEM((_DR, _L), jnp.float32),
            pltpu.VMEM((_DB, _B), jnp.int32),
            pltpu.VMEM_SHARED((_NP, F), jnp.float32),
            pltpu.VMEM_SHARED((_DR, _L), jnp.float32),
            pltpu.SemaphoreType.DMA,
            pltpu.SemaphoreType.DMA,
            pltpu.SemaphoreType.DMA,
            pltpu.SemaphoreType.DMA,
        ],
    )
    def sc_edge(src3, dst3, as_t, ad_t, h, zrows, zden, ident, mout, dout,
                src_b, dst_b, as_b, ad_b, gbuf0, gbuf1, sbuf0, sbuf1, pbuf,
                dloc, ident_b, acc, dacc, sem_g0, sem_g1, sem_s0, sem_s1):
        c = lax.axis_index("c")
        s = lax.axis_index("s")
        wid = c * _NSUB + s
        pltpu.sync_copy(src3.at[wid], src_b)
        pltpu.sync_copy(dst3.at[wid], dst_b)
        pltpu.sync_copy(as_t, as_b)
        pltpu.sync_copy(ad_t, ad_b)
        pltpu.sync_copy(ident, ident_b)
        pltpu.sync_copy(zden, dloc)
        pltpu.sync_copy(zrows, acc.at[pl.ds(s * _RPT, _RPT)])
        pltpu.sync_copy(zden.at[pl.ds(0, _DRT)], dacc.at[pl.ds(s * _DRT, _DRT)])
        plsc.subcore_barrier()

        def compute_p(j):
            for g in range(_B // _L):
                sv = src_b[j, pl.ds(g * _L, _L)]
                dv = dst_b[j, pl.ds(g * _L, _L)]
                u = plsc.load_gather(as_b, [sv]) + plsc.load_gather(ad_b, [dv])
                p = jnp.exp(jnp.maximum(u, 0.2 * u))
                pbuf[pl.ds(g * _L, _L)] = p
                plsc.addupdate_scatter(
                    dloc,
                    [lax.shift_right_logical(dv, 4), lax.bitwise_and(dv, 15)],
                    p)

        def scale(gb, sb):
            for g in range(_B // _L):
                for k in range(_L):
                    e = g * _L + k
                    pk = plsc.load_gather(pbuf, [jnp.full((_L,), e, jnp.int32)])
                    for q in range(F // _L):
                        sb[e, pl.ds(q * _L, _L)] = gb[e, pl.ds(q * _L, _L)] * pk

        _TH = _CH // 2

        def body(t, carry):
            j0 = 2 * t
            j1 = j0 + 1
            cp0 = pltpu.async_copy(h.at[src_b.at[j0]], gbuf0, sem_g0)
            cp1 = pltpu.async_copy(h.at[src_b.at[j1]], gbuf1, sem_g1)
            compute_p(j0)
            cp0.wait()
            scale(gbuf0, sbuf0)
            cs0 = pltpu.async_copy(sbuf0, acc.at[dst_b.at[j0]], sem_s0, add=True)
            compute_p(j1)
            cp1.wait()
            scale(gbuf1, sbuf1)
            cs1 = pltpu.async_copy(sbuf1, acc.at[dst_b.at[j1]], sem_s1, add=True)
            cs0.wait()
            cs1.wait()
            return carry

        lax.fori_loop(0, _TH, body, 0)
        for r in range(_DB):
            pltpu.sync_copy(dloc.at[pl.ds(r * _B, _B)],
                            dacc.at[ident_b.at[r]], add=True)
        plsc.subcore_barrier()
        pltpu.sync_copy(acc.at[pl.ds(s * _RPT, _RPT)],
                        mout.at[c, pl.ds(s * _RPT, _RPT)])
        pltpu.sync_copy(dacc.at[pl.ds(s * _DRT, _DRT)],
                        dout.at[c, pl.ds(s * _DRT, _DRT)])

    return sc_edge


_sc_edge_32 = _make_sc_edge(32)
_sc_edge_64 = _make_sc_edge(64)


def kernel(x, edge_index, W1, a1s, a1d, b1, W2, a2s, a2d, b2, Wres, bres, Wfc, bfc):
    f32 = jnp.float32
    xp = jnp.pad(x.astype(f32), ((0, _NP - _N), (0, 0)))

    loop = jnp.arange(_N, dtype=jnp.int32)
    npad = _EPAD - _E - _N
    src = jnp.concatenate([edge_index[0].astype(jnp.int32), loop,
                           jnp.zeros((npad,), jnp.int32)]).reshape(_NW, _CH, _B)
    dst = jnp.concatenate([edge_index[1].astype(jnp.int32), loop,
                           jnp.full((npad,), _N, jnp.int32)]).reshape(_NW, _CH, _B)
    ident = jnp.arange(_DR, dtype=jnp.int32).reshape(_DB, _B)

    z32 = jnp.zeros((_RPT, 32), f32)
    z64 = jnp.zeros((_RPT, 64), f32)
    zden = jnp.zeros((_DR, _L), f32)

    h1, as1, ad1, res = _tc1(xp, W1, a1s.reshape(32, 1), a1d.reshape(32, 1),
                             Wres, bres.reshape(1, 64))
    m1, d1 = _sc_edge_32(src, dst, as1[:_NT, 0], ad1[:_NT, 0], h1, z32, zden, ident)
    h2, as2, ad2 = _tc2(m1[0], m1[1],
                        d1[0].reshape(_NP, 1), d1[1].reshape(_NP, 1),
                        b1.reshape(1, 32), W2,
                        a2s.reshape(64, 1), a2d.reshape(64, 1))
    m2, d2 = _sc_edge_64(src, dst, as2[:_NT, 0], ad2[:_NT, 0], h2, z64, zden, ident)
    out = _tc3(m2[0], m2[1],
               d2[0].reshape(_NP, 1), d2[1].reshape(_NP, 1),
               b2.reshape(1, 64), res, Wfc, bfc.reshape(1, 1))
    return out[:_N]


# speedup vs baseline: 1.3011x; 1.3011x over previous
import functools

import jax
import jax.numpy as jnp
from jax import lax
from jax.experimental import pallas as pl
from jax.experimental.pallas import tpu as pltpu
from jax.experimental.pallas import tpu_sc as plsc

_N = 10000
_D = 128
_E = 320000

_L = 16
_NSUB = 16
_NCORE = 2
_NW = _NCORE * _NSUB
_B = 128
_CH = 82
_EPW = _B * _CH
_EPAD = _NW * _EPW
_NP = 10240
_NT = 10016
_RPT = _NP // _NSUB
_DR = _NP // _L
_DB = _DR // _B
_DRT = _DR // _NSUB

_ROWS = 1024


def _tc1_body(x_ref, w1_ref, a1s_ref, a1d_ref, wres_ref, bres_ref,
              h1_ref, as_ref, ad_ref, res_ref):
    xb = x_ref[...]
    h = jnp.dot(xb, w1_ref[...], preferred_element_type=jnp.float32)
    h1_ref[...] = h.astype(jnp.bfloat16)
    as_ref[...] = jnp.dot(h, a1s_ref[...], preferred_element_type=jnp.float32)
    ad_ref[...] = jnp.dot(h, a1d_ref[...], preferred_element_type=jnp.float32)
    res_ref[...] = (jnp.dot(xb, wres_ref[...], preferred_element_type=jnp.float32)
                    + bres_ref[...])


def _tc1(xp, W1, a1s, a1d, Wres, bres):
    g = _NP // _ROWS
    return pl.pallas_call(
        _tc1_body,
        grid=(g,),
        in_specs=[
            pl.BlockSpec((_ROWS, _D), lambda i: (i, 0)),
            pl.BlockSpec((_D, 32), lambda i: (0, 0)),
            pl.BlockSpec((32, 1), lambda i: (0, 0)),
            pl.BlockSpec((32, 1), lambda i: (0, 0)),
            pl.BlockSpec((_D, 64), lambda i: (0, 0)),
            pl.BlockSpec((1, 64), lambda i: (0, 0)),
        ],
        out_specs=[
            pl.BlockSpec((_ROWS, 32), lambda i: (i, 0)),
            pl.BlockSpec((_ROWS, 1), lambda i: (i, 0)),
            pl.BlockSpec((_ROWS, 1), lambda i: (i, 0)),
            pl.BlockSpec((_ROWS, 64), lambda i: (i, 0)),
        ],
        out_shape=[
            jax.ShapeDtypeStruct((_NP, 32), jnp.bfloat16),
            jax.ShapeDtypeStruct((_NP, 1), jnp.float32),
            jax.ShapeDtypeStruct((_NP, 1), jnp.float32),
            jax.ShapeDtypeStruct((_NP, 64), jnp.float32),
        ],
    )(xp, W1, a1s, a1d, Wres, bres)


def _tc2_body(m0_ref, m1_ref, d0_ref, d1_ref, b1_ref, w2_ref, a2s_ref, a2d_ref,
              h2_ref, as_ref, ad_ref):
    num = (m0_ref[...].astype(jnp.float32) + m1_ref[...].astype(jnp.float32))
    den = d0_ref[...] + d1_ref[...] + 1e-16
    z = jnp.maximum(num / den + b1_ref[...], 0.0)
    h2 = jnp.dot(z, w2_ref[...], preferred_element_type=jnp.float32)
    h2_ref[...] = h2.astype(jnp.bfloat16)
    as_ref[...] = jnp.dot(h2, a2s_ref[...], preferred_element_type=jnp.float32)
    ad_ref[...] = jnp.dot(h2, a2d_ref[...], preferred_element_type=jnp.float32)


def _tc2(m0, m1, d0, d1, b1, W2, a2s, a2d):
    g = _NP // _ROWS
    return pl.pallas_call(
        _tc2_body,
        grid=(g,),
        in_specs=[
            pl.BlockSpec((_ROWS, 32), lambda i: (i, 0)),
            pl.BlockSpec((_ROWS, 32), lambda i: (i, 0)),
            pl.BlockSpec((_ROWS, 1), lambda i: (i, 0)),
            pl.BlockSpec((_ROWS, 1), lambda i: (i, 0)),
            pl.BlockSpec((1, 32), lambda i: (0, 0)),
            pl.BlockSpec((32, 64), lambda i: (0, 0)),
            pl.BlockSpec((64, 1), lambda i: (0, 0)),
            pl.BlockSpec((64, 1), lambda i: (0, 0)),
        ],
        out_specs=[
            pl.BlockSpec((_ROWS, 64), lambda i: (i, 0)),
            pl.BlockSpec((_ROWS, 1), lambda i: (i, 0)),
            pl.BlockSpec((_ROWS, 1), lambda i: (i, 0)),
        ],
        out_shape=[
            jax.ShapeDtypeStruct((_NP, 64), jnp.bfloat16),
            jax.ShapeDtypeStruct((_NP, 1), jnp.float32),
            jax.ShapeDtypeStruct((_NP, 1), jnp.float32),
        ],
    )(m0, m1, d0, d1, b1, W2, a2s, a2d)


def _tc3_body(m0_ref, m1_ref, d0_ref, d1_ref, b2_ref, res_ref, wfc_ref, bfc_ref,
              o_ref):
    num = (m0_ref[...].astype(jnp.float32) + m1_ref[...].astype(jnp.float32))
    den = d0_ref[...] + d1_ref[...] + 1e-16
    z = jnp.maximum(num / den + b2_ref[...], 0.0)
    t = z + res_ref[...]
    o_ref[...] = jax.nn.sigmoid(
        jnp.dot(t, wfc_ref[...], preferred_element_type=jnp.float32) + bfc_ref[...])


def _tc3(m0, m1, d0, d1, b2, res, Wfc, bfc):
    g = _NP // _ROWS
    return pl.pallas_call(
        _tc3_body,
        grid=(g,),
        in_specs=[
            pl.BlockSpec((_ROWS, 64), lambda i: (i, 0)),
            pl.BlockSpec((_ROWS, 64), lambda i: (i, 0)),
            pl.BlockSpec((_ROWS, 1), lambda i: (i, 0)),
            pl.BlockSpec((_ROWS, 1), lambda i: (i, 0)),
            pl.BlockSpec((1, 64), lambda i: (0, 0)),
            pl.BlockSpec((_ROWS, 64), lambda i: (i, 0)),
            pl.BlockSpec((64, 1), lambda i: (0, 0)),
            pl.BlockSpec((1, 1), lambda i: (0, 0)),
        ],
        out_specs=pl.BlockSpec((_ROWS, 1), lambda i: (i, 0)),
        out_shape=jax.ShapeDtypeStruct((_NP, 1), jnp.float32),
    )(m0, m1, d0, d1, b2, res, Wfc, bfc)


def _make_sc_edge(F):
    mesh = plsc.VectorSubcoreMesh(core_axis_name="c", subcore_axis_name="s")

    @functools.partial(
        pl.kernel,
        out_type=[
            jax.ShapeDtypeStruct((_NCORE, _NP, F), jnp.bfloat16),
            jax.ShapeDtypeStruct((_NCORE, _DR, _L), jnp.float32),
        ],
        mesh=mesh,
        compiler_params=pltpu.CompilerParams(needs_layout_passes=False,
                                             use_tc_tiling_on_sc=False),
        scratch_types=[
            pltpu.VMEM((_CH, _B), jnp.int32),
            pltpu.VMEM((_CH, _B), jnp.int32),
            pltpu.VMEM((_NT,), jnp.float32),
            pltpu.VMEM((_NT,), jnp.float32),
            pltpu.VMEM((_B, F), jnp.bfloat16),
            pltpu.VMEM((_B, F), jnp.bfloat16),
            pltpu.VMEM((_B, F), jnp.bfloat16),
            pltpu.VMEM((_B, F), jnp.bfloat16),
            pltpu.VMEM((_B,), jnp.float32),
            pltpu.VMEM((_DR, _L), jnp.float32),
            pltpu.VMEM((_DB, _B), jnp.int32),
            pltpu.VMEM_SHARED((_NP, F), jnp.bfloat16),
            pltpu.VMEM_SHARED((_DR, _L), jnp.float32),
            pltpu.SemaphoreType.DMA,
            pltpu.SemaphoreType.DMA,
            pltpu.SemaphoreType.DMA,
            pltpu.SemaphoreType.DMA,
        ],
    )
    def sc_edge(src3, dst3, as_t, ad_t, h, zrows, zden, ident, mout, dout,
                src_b, dst_b, as_b, ad_b, gbuf0, gbuf1, sbuf0, sbuf1, pbuf,
                dloc, ident_b, acc, dacc, sem_g0, sem_g1, sem_s0, sem_s1):
        c = lax.axis_index("c")
        s = lax.axis_index("s")
        wid = c * _NSUB + s
        pltpu.sync_copy(src3.at[wid], src_b)
        pltpu.sync_copy(dst3.at[wid], dst_b)
        pltpu.sync_copy(as_t, as_b)
        pltpu.sync_copy(ad_t, ad_b)
        pltpu.sync_copy(ident, ident_b)
        pltpu.sync_copy(zden, dloc)
        pltpu.sync_copy(zrows, acc.at[pl.ds(s * _RPT, _RPT)])
        pltpu.sync_copy(zden.at[pl.ds(0, _DRT)], dacc.at[pl.ds(s * _DRT, _DRT)])
        plsc.subcore_barrier()

        def compute_p(j):
            for g in range(_B // _L):
                sv = src_b[j, pl.ds(g * _L, _L)]
                dv = dst_b[j, pl.ds(g * _L, _L)]
                u = plsc.load_gather(as_b, [sv]) + plsc.load_gather(ad_b, [dv])
                p = jnp.exp(jnp.maximum(u, 0.2 * u))
                pbuf[pl.ds(g * _L, _L)] = p
                plsc.addupdate_scatter(
                    dloc,
                    [lax.shift_right_logical(dv, 4), lax.bitwise_and(dv, 15)],
                    p)

        def scale(gb, sb):
            for g in range(_B // _L):
                for k in range(_L):
                    e = g * _L + k
                    pk = plsc.load_gather(pbuf, [jnp.full((_L,), e, jnp.int32)])
                    pkb = plsc.pack(pk, pk, format=plsc.PackFormat.INTERLEAVED)
                    for q in range(F // (2 * _L)):
                        sb[e, pl.ds(q * 2 * _L, 2 * _L)] = (
                            gb[e, pl.ds(q * 2 * _L, 2 * _L)] * pkb)

        _TH = _CH // 2

        def body(t, carry):
            j0 = 2 * t
            j1 = j0 + 1
            cp0 = pltpu.async_copy(h.at[src_b.at[j0]], gbuf0, sem_g0)
            cp1 = pltpu.async_copy(h.at[src_b.at[j1]], gbuf1, sem_g1)
            compute_p(j0)
            cp0.wait()
            scale(gbuf0, sbuf0)
            cs0 = pltpu.async_copy(sbuf0, acc.at[dst_b.at[j0]], sem_s0, add=True)
            compute_p(j1)
            cp1.wait()
            scale(gbuf1, sbuf1)
            cs1 = pltpu.async_copy(sbuf1, acc.at[dst_b.at[j1]], sem_s1, add=True)
            cs0.wait()
            cs1.wait()
            return carry

        lax.fori_loop(0, _TH, body, 0)
        for r in range(_DB):
            pltpu.sync_copy(dloc.at[pl.ds(r * _B, _B)],
                            dacc.at[ident_b.at[r]], add=True)
        plsc.subcore_barrier()
        pltpu.sync_copy(acc.at[pl.ds(s * _RPT, _RPT)],
                        mout.at[c, pl.ds(s * _RPT, _RPT)])
        pltpu.sync_copy(dacc.at[pl.ds(s * _DRT, _DRT)],
                        dout.at[c, pl.ds(s * _DRT, _DRT)])

    return sc_edge


_sc_edge_32 = _make_sc_edge(32)
_sc_edge_64 = _make_sc_edge(64)


def kernel(x, edge_index, W1, a1s, a1d, b1, W2, a2s, a2d, b2, Wres, bres, Wfc, bfc):
    f32 = jnp.float32
    xp = jnp.pad(x.astype(f32), ((0, _NP - _N), (0, 0)))

    loop = jnp.arange(_N, dtype=jnp.int32)
    npad = _EPAD - _E - _N
    src = jnp.concatenate([edge_index[0].astype(jnp.int32), loop,
                           jnp.zeros((npad,), jnp.int32)]).reshape(_NW, _CH, _B)
    dst = jnp.concatenate([edge_index[1].astype(jnp.int32), loop,
                           jnp.full((npad,), _N, jnp.int32)]).reshape(_NW, _CH, _B)
    ident = jnp.arange(_DR, dtype=jnp.int32).reshape(_DB, _B)

    z32 = jnp.zeros((_RPT, 32), jnp.bfloat16)
    z64 = jnp.zeros((_RPT, 64), jnp.bfloat16)
    zden = jnp.zeros((_DR, _L), f32)

    h1, as1, ad1, res = _tc1(xp, W1, a1s.reshape(32, 1), a1d.reshape(32, 1),
                             Wres, bres.reshape(1, 64))
    m1, d1 = _sc_edge_32(src, dst, as1[:_NT, 0], ad1[:_NT, 0], h1, z32, zden, ident)
    h2, as2, ad2 = _tc2(m1[0], m1[1],
                        d1[0].reshape(_NP, 1), d1[1].reshape(_NP, 1),
                        b1.reshape(1, 32), W2,
                        a2s.reshape(64, 1), a2d.reshape(64, 1))
    m2, d2 = _sc_edge_64(src, dst, as2[:_NT, 0], ad2[:_NT, 0], h2, z64, zden, ident)
    out = _tc3(m2[0], m2[1],
               d2[0].reshape(_NP, 1), d2[1].reshape(_NP, 1),
               b2.reshape(1, 64), res, Wfc, bfc.reshape(1, 1))
    return out[:_N]


# speedup vs baseline: 1.4929x; 1.1474x over previous
import functools

import jax
import jax.numpy as jnp
from jax import lax
from jax.experimental import pallas as pl
from jax.experimental.pallas import tpu as pltpu
from jax.experimental.pallas import tpu_sc as plsc

_N = 10000
_D = 128
_E = 320000

_L = 16
_NSUB = 16
_NCORE = 2
_NW = _NCORE * _NSUB
_B = 256
_CH = 42
_EPW = _B * _CH
_EPAD = _NW * _EPW
_NP = 10240
_NT = 10016
_RPT = _NP // _NSUB
_DR = _NP // _L
_MB = 128
_DB = _DR // _MB
_DRT = _DR // _NSUB

_ROWS = 1024


def _tc1_body(x_ref, w1_ref, a1s_ref, a1d_ref, wres_ref, bres_ref,
              h1_ref, as_ref, ad_ref, res_ref):
    xb = x_ref[...]
    h = jnp.dot(xb, w1_ref[...], preferred_element_type=jnp.float32)
    h1_ref[...] = h.astype(jnp.bfloat16)
    as_ref[...] = jnp.dot(h, a1s_ref[...], preferred_element_type=jnp.float32)
    ad_ref[...] = jnp.dot(h, a1d_ref[...], preferred_element_type=jnp.float32)
    res_ref[...] = (jnp.dot(xb, wres_ref[...], preferred_element_type=jnp.float32)
                    + bres_ref[...])


def _tc1(xp, W1, a1s, a1d, Wres, bres):
    g = _NP // _ROWS
    return pl.pallas_call(
        _tc1_body,
        grid=(g,),
        in_specs=[
            pl.BlockSpec((_ROWS, _D), lambda i: (i, 0)),
            pl.BlockSpec((_D, 32), lambda i: (0, 0)),
            pl.BlockSpec((32, 1), lambda i: (0, 0)),
            pl.BlockSpec((32, 1), lambda i: (0, 0)),
            pl.BlockSpec((_D, 64), lambda i: (0, 0)),
            pl.BlockSpec((1, 64), lambda i: (0, 0)),
        ],
        out_specs=[
            pl.BlockSpec((_ROWS, 32), lambda i: (i, 0)),
            pl.BlockSpec((_ROWS, 1), lambda i: (i, 0)),
            pl.BlockSpec((_ROWS, 1), lambda i: (i, 0)),
            pl.BlockSpec((_ROWS, 64), lambda i: (i, 0)),
        ],
        out_shape=[
            jax.ShapeDtypeStruct((_NP, 32), jnp.bfloat16),
            jax.ShapeDtypeStruct((_NP, 1), jnp.float32),
            jax.ShapeDtypeStruct((_NP, 1), jnp.float32),
            jax.ShapeDtypeStruct((_NP, 64), jnp.float32),
        ],
    )(xp, W1, a1s, a1d, Wres, bres)


def _tc2_body(m0_ref, m1_ref, d0_ref, d1_ref, b1_ref, w2_ref, a2s_ref, a2d_ref,
              h2_ref, as_ref, ad_ref):
    num = (m0_ref[...].astype(jnp.float32) + m1_ref[...].astype(jnp.float32))
    den = d0_ref[...] + d1_ref[...] + 1e-16
    z = jnp.maximum(num / den + b1_ref[...], 0.0)
    h2 = jnp.dot(z, w2_ref[...], preferred_element_type=jnp.float32)
    h2_ref[...] = h2.astype(jnp.bfloat16)
    as_ref[...] = jnp.dot(h2, a2s_ref[...], preferred_element_type=jnp.float32)
    ad_ref[...] = jnp.dot(h2, a2d_ref[...], preferred_element_type=jnp.float32)


def _tc2(m0, m1, d0, d1, b1, W2, a2s, a2d):
    g = _NP // _ROWS
    return pl.pallas_call(
        _tc2_body,
        grid=(g,),
        in_specs=[
            pl.BlockSpec((_ROWS, 32), lambda i: (i, 0)),
            pl.BlockSpec((_ROWS, 32), lambda i: (i, 0)),
            pl.BlockSpec((_ROWS, 1), lambda i: (i, 0)),
            pl.BlockSpec((_ROWS, 1), lambda i: (i, 0)),
            pl.BlockSpec((1, 32), lambda i: (0, 0)),
            pl.BlockSpec((32, 64), lambda i: (0, 0)),
            pl.BlockSpec((64, 1), lambda i: (0, 0)),
            pl.BlockSpec((64, 1), lambda i: (0, 0)),
        ],
        out_specs=[
            pl.BlockSpec((_ROWS, 64), lambda i: (i, 0)),
            pl.BlockSpec((_ROWS, 1), lambda i: (i, 0)),
            pl.BlockSpec((_ROWS, 1), lambda i: (i, 0)),
        ],
        out_shape=[
            jax.ShapeDtypeStruct((_NP, 64), jnp.bfloat16),
            jax.ShapeDtypeStruct((_NP, 1), jnp.float32),
            jax.ShapeDtypeStruct((_NP, 1), jnp.float32),
        ],
    )(m0, m1, d0, d1, b1, W2, a2s, a2d)


def _tc3_body(m0_ref, m1_ref, d0_ref, d1_ref, b2_ref, res_ref, wfc_ref, bfc_ref,
              o_ref):
    num = (m0_ref[...].astype(jnp.float32) + m1_ref[...].astype(jnp.float32))
    den = d0_ref[...] + d1_ref[...] + 1e-16
    z = jnp.maximum(num / den + b2_ref[...], 0.0)
    t = z + res_ref[...]
    o_ref[...] = jax.nn.sigmoid(
        jnp.dot(t, wfc_ref[...], preferred_element_type=jnp.float32) + bfc_ref[...])


def _tc3(m0, m1, d0, d1, b2, res, Wfc, bfc):
    g = _NP // _ROWS
    return pl.pallas_call(
        _tc3_body,
        grid=(g,),
        in_specs=[
            pl.BlockSpec((_ROWS, 64), lambda i: (i, 0)),
            pl.BlockSpec((_ROWS, 64), lambda i: (i, 0)),
            pl.BlockSpec((_ROWS, 1), lambda i: (i, 0)),
            pl.BlockSpec((_ROWS, 1), lambda i: (i, 0)),
            pl.BlockSpec((1, 64), lambda i: (0, 0)),
            pl.BlockSpec((_ROWS, 64), lambda i: (i, 0)),
            pl.BlockSpec((64, 1), lambda i: (0, 0)),
            pl.BlockSpec((1, 1), lambda i: (0, 0)),
        ],
        out_specs=pl.BlockSpec((_ROWS, 1), lambda i: (i, 0)),
        out_shape=jax.ShapeDtypeStruct((_NP, 1), jnp.float32),
    )(m0, m1, d0, d1, b2, res, Wfc, bfc)


def _make_sc_edge(F):
    mesh = plsc.VectorSubcoreMesh(core_axis_name="c", subcore_axis_name="s")

    @functools.partial(
        pl.kernel,
        out_type=[
            jax.ShapeDtypeStruct((_NCORE, _NP, F), jnp.bfloat16),
            jax.ShapeDtypeStruct((_NCORE, _DR, _L), jnp.float32),
        ],
        mesh=mesh,
        compiler_params=pltpu.CompilerParams(needs_layout_passes=False,
                                             use_tc_tiling_on_sc=False),
        scratch_types=[
            pltpu.VMEM((_CH, _B), jnp.int32),
            pltpu.VMEM((_CH, _B), jnp.int32),
            pltpu.VMEM((_NT,), jnp.float32),
            pltpu.VMEM((_NT,), jnp.float32),
            pltpu.VMEM((_B, F), jnp.bfloat16),
            pltpu.VMEM((_B, F), jnp.bfloat16),
            pltpu.VMEM((_B, F), jnp.bfloat16),
            pltpu.VMEM((_B, F), jnp.bfloat16),
            pltpu.VMEM((_B,), jnp.float32),
            pltpu.VMEM((_DR, _L), jnp.float32),
            pltpu.VMEM((_DB, _MB), jnp.int32),
            pltpu.VMEM_SHARED((_NP, F), jnp.bfloat16),
            pltpu.VMEM_SHARED((_DR, _L), jnp.float32),
            pltpu.SemaphoreType.DMA,
            pltpu.SemaphoreType.DMA,
            pltpu.SemaphoreType.DMA,
            pltpu.SemaphoreType.DMA,
        ],
    )
    def sc_edge(src3, dst3, as_t, ad_t, h, zrows, zden, ident, mout, dout,
                src_b, dst_b, as_b, ad_b, gbuf0, gbuf1, sbuf0, sbuf1, pbuf,
                dloc, ident_b, acc, dacc, sem_g0, sem_g1, sem_s0, sem_s1):
        c = lax.axis_index("c")
        s = lax.axis_index("s")
        wid = c * _NSUB + s
        pltpu.sync_copy(src3.at[wid], src_b)
        pltpu.sync_copy(dst3.at[wid], dst_b)
        pltpu.sync_copy(as_t, as_b)
        pltpu.sync_copy(ad_t, ad_b)
        pltpu.sync_copy(ident, ident_b)
        pltpu.sync_copy(zden, dloc)
        pltpu.sync_copy(zrows, acc.at[pl.ds(s * _RPT, _RPT)])
        pltpu.sync_copy(zden.at[pl.ds(0, _DRT)], dacc.at[pl.ds(s * _DRT, _DRT)])
        plsc.subcore_barrier()

        def compute_p(j):
            for g in range(_B // _L):
                sv = src_b[j, pl.ds(g * _L, _L)]
                dv = dst_b[j, pl.ds(g * _L, _L)]
                u = plsc.load_gather(as_b, [sv]) + plsc.load_gather(ad_b, [dv])
                p = jnp.exp(jnp.maximum(u, 0.2 * u))
                pbuf[pl.ds(g * _L, _L)] = p
                plsc.addupdate_scatter(
                    dloc,
                    [lax.shift_right_logical(dv, 4), lax.bitwise_and(dv, 15)],
                    p)

        def scale(gb, sb):
            for g in range(_B // _L):
                pg = pbuf[pl.ds(g * _L, _L)]
                for k in range(_L):
                    e = g * _L + k
                    pk = lax.gather(
                        pg, jnp.full((_L, 1), k, jnp.int32),
                        lax.GatherDimensionNumbers(
                            offset_dims=(), collapsed_slice_dims=(0,),
                            start_index_map=(0,)),
                        slice_sizes=(1,),
                        mode=lax.GatherScatterMode.PROMISE_IN_BOUNDS)
                    pkb = plsc.pack(pk, pk, format=plsc.PackFormat.INTERLEAVED)
                    for q in range(F // (2 * _L)):
                        sb[e, pl.ds(q * 2 * _L, 2 * _L)] = (
                            gb[e, pl.ds(q * 2 * _L, 2 * _L)] * pkb)

        _TH = _CH // 2

        def body(t, carry):
            j0 = 2 * t
            j1 = j0 + 1
            cp0 = pltpu.async_copy(h.at[src_b.at[j0]], gbuf0, sem_g0)
            cp1 = pltpu.async_copy(h.at[src_b.at[j1]], gbuf1, sem_g1)
            compute_p(j0)
            cp0.wait()
            scale(gbuf0, sbuf0)
            cs0 = pltpu.async_copy(sbuf0, acc.at[dst_b.at[j0]], sem_s0, add=True)
            compute_p(j1)
            cp1.wait()
            scale(gbuf1, sbuf1)
            cs1 = pltpu.async_copy(sbuf1, acc.at[dst_b.at[j1]], sem_s1, add=True)
            cs0.wait()
            cs1.wait()
            return carry

        lax.fori_loop(0, _TH, body, 0)
        for r in range(_DB):
            pltpu.sync_copy(dloc.at[pl.ds(r * _MB, _MB)],
                            dacc.at[ident_b.at[r]], add=True)
        plsc.subcore_barrier()
        pltpu.sync_copy(acc.at[pl.ds(s * _RPT, _RPT)],
                        mout.at[c, pl.ds(s * _RPT, _RPT)])
        pltpu.sync_copy(dacc.at[pl.ds(s * _DRT, _DRT)],
                        dout.at[c, pl.ds(s * _DRT, _DRT)])

    return sc_edge


_sc_edge_32 = _make_sc_edge(32)
_sc_edge_64 = _make_sc_edge(64)


def kernel(x, edge_index, W1, a1s, a1d, b1, W2, a2s, a2d, b2, Wres, bres, Wfc, bfc):
    f32 = jnp.float32
    xp = jnp.pad(x.astype(f32), ((0, _NP - _N), (0, 0)))

    loop = jnp.arange(_N, dtype=jnp.int32)
    npad = _EPAD - _E - _N
    src = jnp.concatenate([edge_index[0].astype(jnp.int32), loop,
                           jnp.zeros((npad,), jnp.int32)]).reshape(_NW, _CH, _B)
    dst = jnp.concatenate([edge_index[1].astype(jnp.int32), loop,
                           jnp.full((npad,), _N, jnp.int32)]).reshape(_NW, _CH, _B)
    ident = jnp.arange(_DR, dtype=jnp.int32).reshape(_DB, _MB)

    z32 = jnp.zeros((_RPT, 32), jnp.bfloat16)
    z64 = jnp.zeros((_RPT, 64), jnp.bfloat16)
    zden = jnp.zeros((_DR, _L), f32)

    h1, as1, ad1, res = _tc1(xp, W1, a1s.reshape(32, 1), a1d.reshape(32, 1),
                             Wres, bres.reshape(1, 64))
    m1, d1 = _sc_edge_32(src, dst, as1[:_NT, 0], ad1[:_NT, 0], h1, z32, zden, ident)
    h2, as2, ad2 = _tc2(m1[0], m1[1],
                        d1[0].reshape(_NP, 1), d1[1].reshape(_NP, 1),
                        b1.reshape(1, 32), W2,
                        a2s.reshape(64, 1), a2d.reshape(64, 1))
    m2, d2 = _sc_edge_64(src, dst, as2[:_NT, 0], ad2[:_NT, 0], h2, z64, zden, ident)
    out = _tc3(m2[0], m2[1],
               d2[0].reshape(_NP, 1), d2[1].reshape(_NP, 1),
               b2.reshape(1, 64), res, Wfc, bfc.reshape(1, 1))
    return out[:_N]


# speedup vs baseline: 1.5168x; 1.0160x over previous
import functools

import jax
import jax.numpy as jnp
from jax import lax
from jax.experimental import pallas as pl
from jax.experimental.pallas import tpu as pltpu
from jax.experimental.pallas import tpu_sc as plsc

_N = 10000
_D = 128
_E = 320000

_L = 16
_NSUB = 16
_NCORE = 2
_NW = _NCORE * _NSUB
_B = 256
_CH = 42
_EPW = _B * _CH
_EPAD = _NW * _EPW
_NP = 10240
_NT = 10016
_RPT = _NP // _NSUB
_DR = _NP // _L
_MB = 128
_DB = _DR // _MB
_DRT = _DR // _NSUB

_ROWS = 1024


def _tc1_body(x_ref, w1_ref, a1s_ref, a1d_ref, wres_ref, bres_ref,
              h1_ref, as_ref, ad_ref, res_ref):
    xb = x_ref[...]
    h = jnp.dot(xb, w1_ref[...], preferred_element_type=jnp.float32)
    h1_ref[...] = h.astype(jnp.bfloat16)
    as_ref[...] = jnp.dot(h, a1s_ref[...], preferred_element_type=jnp.float32)
    ad_ref[...] = jnp.dot(h, a1d_ref[...], preferred_element_type=jnp.float32)
    res_ref[...] = (jnp.dot(xb, wres_ref[...], preferred_element_type=jnp.float32)
                    + bres_ref[...])


def _tc1(xp, W1, a1s, a1d, Wres, bres):
    g = _NP // _ROWS
    return pl.pallas_call(
        _tc1_body,
        grid=(g,),
        in_specs=[
            pl.BlockSpec((_ROWS, _D), lambda i: (i, 0)),
            pl.BlockSpec((_D, 32), lambda i: (0, 0)),
            pl.BlockSpec((32, 1), lambda i: (0, 0)),
            pl.BlockSpec((32, 1), lambda i: (0, 0)),
            pl.BlockSpec((_D, 64), lambda i: (0, 0)),
            pl.BlockSpec((1, 64), lambda i: (0, 0)),
        ],
        out_specs=[
            pl.BlockSpec((_ROWS, 32), lambda i: (i, 0)),
            pl.BlockSpec((_ROWS, 1), lambda i: (i, 0)),
            pl.BlockSpec((_ROWS, 1), lambda i: (i, 0)),
            pl.BlockSpec((_ROWS, 64), lambda i: (i, 0)),
        ],
        out_shape=[
            jax.ShapeDtypeStruct((_NP, 32), jnp.bfloat16),
            jax.ShapeDtypeStruct((_NP, 1), jnp.float32),
            jax.ShapeDtypeStruct((_NP, 1), jnp.float32),
            jax.ShapeDtypeStruct((_NP, 64), jnp.float32),
        ],
    )(xp, W1, a1s, a1d, Wres, bres)


def _tc2_body(m0_ref, m1_ref, d0_ref, d1_ref, b1_ref, w2_ref, a2s_ref, a2d_ref,
              h2_ref, as_ref, ad_ref):
    num = (m0_ref[...].astype(jnp.float32) + m1_ref[...].astype(jnp.float32))
    den = d0_ref[...] + d1_ref[...] + 1e-16
    z = jnp.maximum(num / den + b1_ref[...], 0.0)
    h2 = jnp.dot(z, w2_ref[...], preferred_element_type=jnp.float32)
    h2_ref[...] = h2.astype(jnp.bfloat16)
    as_ref[...] = jnp.dot(h2, a2s_ref[...], preferred_element_type=jnp.float32)
    ad_ref[...] = jnp.dot(h2, a2d_ref[...], preferred_element_type=jnp.float32)


def _tc2(m0, m1, d0, d1, b1, W2, a2s, a2d):
    g = _NP // _ROWS
    return pl.pallas_call(
        _tc2_body,
        grid=(g,),
        in_specs=[
            pl.BlockSpec((_ROWS, 32), lambda i: (i, 0)),
            pl.BlockSpec((_ROWS, 32), lambda i: (i, 0)),
            pl.BlockSpec((_ROWS, 1), lambda i: (i, 0)),
            pl.BlockSpec((_ROWS, 1), lambda i: (i, 0)),
            pl.BlockSpec((1, 32), lambda i: (0, 0)),
            pl.BlockSpec((32, 64), lambda i: (0, 0)),
            pl.BlockSpec((64, 1), lambda i: (0, 0)),
            pl.BlockSpec((64, 1), lambda i: (0, 0)),
        ],
        out_specs=[
            pl.BlockSpec((_ROWS, 64), lambda i: (i, 0)),
            pl.BlockSpec((_ROWS, 1), lambda i: (i, 0)),
            pl.BlockSpec((_ROWS, 1), lambda i: (i, 0)),
        ],
        out_shape=[
            jax.ShapeDtypeStruct((_NP, 64), jnp.bfloat16),
            jax.ShapeDtypeStruct((_NP, 1), jnp.float32),
            jax.ShapeDtypeStruct((_NP, 1), jnp.float32),
        ],
    )(m0, m1, d0, d1, b1, W2, a2s, a2d)


def _tc3_body(m0_ref, m1_ref, d0_ref, d1_ref, b2_ref, res_ref, wfc_ref, bfc_ref,
              o_ref):
    num = (m0_ref[...].astype(jnp.float32) + m1_ref[...].astype(jnp.float32))
    den = d0_ref[...] + d1_ref[...] + 1e-16
    z = jnp.maximum(num / den + b2_ref[...], 0.0)
    t = z + res_ref[...]
    o_ref[...] = jax.nn.sigmoid(
        jnp.dot(t, wfc_ref[...], preferred_element_type=jnp.float32) + bfc_ref[...])


def _tc3(m0, m1, d0, d1, b2, res, Wfc, bfc):
    g = _NP // _ROWS
    return pl.pallas_call(
        _tc3_body,
        grid=(g,),
        in_specs=[
            pl.BlockSpec((_ROWS, 64), lambda i: (i, 0)),
            pl.BlockSpec((_ROWS, 64), lambda i: (i, 0)),
            pl.BlockSpec((_ROWS, 1), lambda i: (i, 0)),
            pl.BlockSpec((_ROWS, 1), lambda i: (i, 0)),
            pl.BlockSpec((1, 64), lambda i: (0, 0)),
            pl.BlockSpec((_ROWS, 64), lambda i: (i, 0)),
            pl.BlockSpec((64, 1), lambda i: (0, 0)),
            pl.BlockSpec((1, 1), lambda i: (0, 0)),
        ],
        out_specs=pl.BlockSpec((_ROWS, 1), lambda i: (i, 0)),
        out_shape=jax.ShapeDtypeStruct((_NP, 1), jnp.float32),
    )(m0, m1, d0, d1, b2, res, Wfc, bfc)


def _make_sc_edge(F):
    mesh = plsc.VectorSubcoreMesh(core_axis_name="c", subcore_axis_name="s")

    @functools.partial(
        pl.kernel,
        out_type=[
            jax.ShapeDtypeStruct((_NCORE, _NP, F), jnp.bfloat16),
            jax.ShapeDtypeStruct((_NCORE, _DR, _L), jnp.float32),
        ],
        mesh=mesh,
        compiler_params=pltpu.CompilerParams(needs_layout_passes=False,
                                             use_tc_tiling_on_sc=False),
        scratch_types=[
            pltpu.VMEM((_CH, _B), jnp.int32),
            pltpu.VMEM((_CH, _B), jnp.int32),
            pltpu.VMEM((_NT,), jnp.float32),
            pltpu.VMEM((_NT,), jnp.float32),
            pltpu.VMEM((_B, F), jnp.bfloat16),
            pltpu.VMEM((_B, F), jnp.bfloat16),
            pltpu.VMEM((_B, F), jnp.bfloat16),
            pltpu.VMEM((_B, F), jnp.bfloat16),
            pltpu.VMEM((_B,), jnp.float32),
            pltpu.VMEM((_DR, _L), jnp.float32),
            pltpu.VMEM((_DB, _MB), jnp.int32),
            pltpu.VMEM_SHARED((_NP, F), jnp.bfloat16),
            pltpu.VMEM_SHARED((_DR, _L), jnp.float32),
            pltpu.SemaphoreType.DMA,
            pltpu.SemaphoreType.DMA,
            pltpu.SemaphoreType.DMA,
            pltpu.SemaphoreType.DMA,
        ],
    )
    def sc_edge(src3, dst3, as_t, ad_t, h, zrows, zden, ident, mout, dout,
                src_b, dst_b, as_b, ad_b, gbuf0, gbuf1, sbuf0, sbuf1, pbuf,
                dloc, ident_b, acc, dacc, sem_g0, sem_g1, sem_s0, sem_s1):
        c = lax.axis_index("c")
        s = lax.axis_index("s")
        wid = c * _NSUB + s
        pltpu.sync_copy(src3.at[wid], src_b)
        pltpu.sync_copy(dst3.at[wid], dst_b)
        pltpu.sync_copy(as_t, as_b)
        pltpu.sync_copy(ad_t, ad_b)
        pltpu.sync_copy(ident, ident_b)
        pltpu.sync_copy(zden, dloc)
        pltpu.sync_copy(zrows, acc.at[pl.ds(s * _RPT, _RPT)])
        pltpu.sync_copy(zden.at[pl.ds(0, _DRT)], dacc.at[pl.ds(s * _DRT, _DRT)])
        plsc.subcore_barrier()

        def compute_p(j):
            for g in range(_B // _L):
                sv = src_b[j, pl.ds(g * _L, _L)]
                dv = dst_b[j, pl.ds(g * _L, _L)]
                u = plsc.load_gather(as_b, [sv]) + plsc.load_gather(ad_b, [dv])
                p = jnp.exp(jnp.maximum(u, 0.2 * u))
                pbuf[pl.ds(g * _L, _L)] = p
                plsc.addupdate_scatter(
                    dloc,
                    [lax.shift_right_logical(dv, 4), lax.bitwise_and(dv, 15)],
                    p)

        def scale(gb, sb):
            for g in range(_B // _L):
                pg = pbuf[pl.ds(g * _L, _L)]
                for k in range(_L):
                    e = g * _L + k
                    pk = lax.gather(
                        pg, jnp.full((_L, 1), k, jnp.int32),
                        lax.GatherDimensionNumbers(
                            offset_dims=(), collapsed_slice_dims=(0,),
                            start_index_map=(0,)),
                        slice_sizes=(1,),
                        mode=lax.GatherScatterMode.PROMISE_IN_BOUNDS)
                    pkb = plsc.pack(pk, pk, format=plsc.PackFormat.INTERLEAVED)
                    for q in range(F // (2 * _L)):
                        sb[e, pl.ds(q * 2 * _L, 2 * _L)] = (
                            gb[e, pl.ds(q * 2 * _L, 2 * _L)] * pkb)

        _TH = _CH // 2

        def body(t, carry):
            j0 = 2 * t
            j1 = j0 + 1
            cp0 = pltpu.async_copy(h.at[src_b.at[j0]], gbuf0, sem_g0)
            cp1 = pltpu.async_copy(h.at[src_b.at[j1]], gbuf1, sem_g1)
            compute_p(j0)
            cp0.wait()
            scale(gbuf0, sbuf0)
            cs0 = pltpu.async_copy(sbuf0, acc.at[dst_b.at[j0]], sem_s0, add=True)
            compute_p(j1)
            cp1.wait()
            scale(gbuf1, sbuf1)
            cs1 = pltpu.async_copy(sbuf1, acc.at[dst_b.at[j1]], sem_s1, add=True)
            cs0.wait()
            cs1.wait()
            return carry

        lax.fori_loop(0, _TH, body, 0)
        for r in range(_DB):
            pltpu.sync_copy(dloc.at[pl.ds(r * _MB, _MB)],
                            dacc.at[ident_b.at[r]], add=True)
        plsc.subcore_barrier()
        pltpu.sync_copy(acc.at[pl.ds(s * _RPT, _RPT)],
                        mout.at[c, pl.ds(s * _RPT, _RPT)])
        pltpu.sync_copy(dacc.at[pl.ds(s * _DRT, _DRT)],
                        dout.at[c, pl.ds(s * _DRT, _DRT)])

    return sc_edge


_sc_edge_32 = _make_sc_edge(32)
_sc_edge_64 = _make_sc_edge(64)


def kernel(x, edge_index, W1, a1s, a1d, b1, W2, a2s, a2d, b2, Wres, bres, Wfc, bfc):
    f32 = jnp.float32
    xp = jnp.pad(x.astype(f32), ((0, _NP - _N), (0, 0)))

    loop = jnp.arange(_N, dtype=jnp.int32)
    npad = _EPAD - _E - _N
    padd = _N + jnp.arange(npad, dtype=jnp.int32) % (_NP - _N)
    src = jnp.concatenate([edge_index[0].astype(jnp.int32), loop,
                           jnp.zeros((npad,), jnp.int32)]).reshape(_NW, _CH, _B)
    dst = jnp.concatenate([edge_index[1].astype(jnp.int32), loop,
                           padd]).reshape(_NW, _CH, _B)
    ident = jnp.arange(_DR, dtype=jnp.int32).reshape(_DB, _MB)

    z32 = jnp.zeros((_RPT, 32), jnp.bfloat16)
    z64 = jnp.zeros((_RPT, 64), jnp.bfloat16)
    zden = jnp.zeros((_DR, _L), f32)

    h1, as1, ad1, res = _tc1(xp, W1, a1s.reshape(32, 1), a1d.reshape(32, 1),
                             Wres, bres.reshape(1, 64))
    m1, d1 = _sc_edge_32(src, dst, as1[:_NT, 0], ad1[:_NT, 0], h1, z32, zden, ident)
    h2, as2, ad2 = _tc2(m1[0], m1[1],
                        d1[0].reshape(_NP, 1), d1[1].reshape(_NP, 1),
                        b1.reshape(1, 32), W2,
                        a2s.reshape(64, 1), a2d.reshape(64, 1))
    m2, d2 = _sc_edge_64(src, dst, as2[:_NT, 0], ad2[:_NT, 0], h2, z64, zden, ident)
    out = _tc3(m2[0], m2[1],
               d2[0].reshape(_NP, 1), d2[1].reshape(_NP, 1),
               b2.reshape(1, 64), res, Wfc, bfc.reshape(1, 1))
    return out[:_N]


# speedup vs baseline: 2.1274x; 1.4026x over previous
import functools

import jax
import jax.numpy as jnp
from jax import lax
from jax.experimental import pallas as pl
from jax.experimental.pallas import tpu as pltpu
from jax.experimental.pallas import tpu_sc as plsc

_N = 10000
_D = 128
_E = 320000

_L = 16
_NSUB = 16
_NCORE = 2
_NW = _NCORE * _NSUB
_B = 256
_CH0 = 58
_CH1 = 24
_CHM = max(_CH0, _CH1)
_TOTCH = _NSUB * (_CH0 + _CH1)
_EPAD = _TOTCH * _B
_XCH = _CHM - min(_CH0, _CH1)
_NP = 10240
_NT = 10016
_RPT = _NP // _NSUB
_DR = _NP // _L
_MB = 128
_DB = _DR // _MB
_DRT = _DR // _NSUB

_ROWS = 1024


def _tc1_body(x_ref, w1_ref, a1s_ref, a1d_ref, wres_ref, bres_ref,
              h1_ref, as_ref, ad_ref, res_ref):
    xb = x_ref[...]
    h = jnp.dot(xb, w1_ref[...], preferred_element_type=jnp.float32)
    h1_ref[...] = h.astype(jnp.bfloat16)
    as_ref[...] = jnp.dot(h, a1s_ref[...], preferred_element_type=jnp.float32)
    ad_ref[...] = jnp.dot(h, a1d_ref[...], preferred_element_type=jnp.float32)
    res_ref[...] = (jnp.dot(xb, wres_ref[...], preferred_element_type=jnp.float32)
                    + bres_ref[...])


def _tc1(xp, W1, a1s, a1d, Wres, bres):
    g = _NP // _ROWS
    return pl.pallas_call(
        _tc1_body,
        grid=(g,),
        in_specs=[
            pl.BlockSpec((_ROWS, _D), lambda i: (i, 0)),
            pl.BlockSpec((_D, 32), lambda i: (0, 0)),
            pl.BlockSpec((32, 1), lambda i: (0, 0)),
            pl.BlockSpec((32, 1), lambda i: (0, 0)),
            pl.BlockSpec((_D, 64), lambda i: (0, 0)),
            pl.BlockSpec((1, 64), lambda i: (0, 0)),
        ],
        out_specs=[
            pl.BlockSpec((_ROWS, 32), lambda i: (i, 0)),
            pl.BlockSpec((_ROWS, 1), lambda i: (i, 0)),
            pl.BlockSpec((_ROWS, 1), lambda i: (i, 0)),
            pl.BlockSpec((_ROWS, 64), lambda i: (i, 0)),
        ],
        out_shape=[
            jax.ShapeDtypeStruct((_NP, 32), jnp.bfloat16),
            jax.ShapeDtypeStruct((_NP, 1), jnp.float32),
            jax.ShapeDtypeStruct((_NP, 1), jnp.float32),
            jax.ShapeDtypeStruct((_NP, 64), jnp.float32),
        ],
    )(xp, W1, a1s, a1d, Wres, bres)


def _tc2_body(m0_ref, m1_ref, d0_ref, d1_ref, b1_ref, w2_ref, a2s_ref, a2d_ref,
              h2_ref, as_ref, ad_ref):
    num = (m0_ref[...].astype(jnp.float32) + m1_ref[...].astype(jnp.float32))
    den = d0_ref[...] + d1_ref[...] + 1e-16
    z = jnp.maximum(num / den + b1_ref[...], 0.0)
    h2 = jnp.dot(z, w2_ref[...], preferred_element_type=jnp.float32)
    h2_ref[...] = h2.astype(jnp.bfloat16)
    as_ref[...] = jnp.dot(h2, a2s_ref[...], preferred_element_type=jnp.float32)
    ad_ref[...] = jnp.dot(h2, a2d_ref[...], preferred_element_type=jnp.float32)


def _tc2(m0, m1, d0, d1, b1, W2, a2s, a2d):
    g = _NP // _ROWS
    return pl.pallas_call(
        _tc2_body,
        grid=(g,),
        in_specs=[
            pl.BlockSpec((_ROWS, 32), lambda i: (i, 0)),
            pl.BlockSpec((_ROWS, 32), lambda i: (i, 0)),
            pl.BlockSpec((_ROWS, 1), lambda i: (i, 0)),
            pl.BlockSpec((_ROWS, 1), lambda i: (i, 0)),
            pl.BlockSpec((1, 32), lambda i: (0, 0)),
            pl.BlockSpec((32, 64), lambda i: (0, 0)),
            pl.BlockSpec((64, 1), lambda i: (0, 0)),
            pl.BlockSpec((64, 1), lambda i: (0, 0)),
        ],
        out_specs=[
            pl.BlockSpec((_ROWS, 64), lambda i: (i, 0)),
            pl.BlockSpec((_ROWS, 1), lambda i: (i, 0)),
            pl.BlockSpec((_ROWS, 1), lambda i: (i, 0)),
        ],
        out_shape=[
            jax.ShapeDtypeStruct((_NP, 64), jnp.bfloat16),
            jax.ShapeDtypeStruct((_NP, 1), jnp.float32),
            jax.ShapeDtypeStruct((_NP, 1), jnp.float32),
        ],
    )(m0, m1, d0, d1, b1, W2, a2s, a2d)


def _tc3_body(m0_ref, m1_ref, d0_ref, d1_ref, b2_ref, res_ref, wfc_ref, bfc_ref,
              o_ref):
    num = (m0_ref[...].astype(jnp.float32) + m1_ref[...].astype(jnp.float32))
    den = d0_ref[...] + d1_ref[...] + 1e-16
    z = jnp.maximum(num / den + b2_ref[...], 0.0)
    t = z + res_ref[...]
    o_ref[...] = jax.nn.sigmoid(
        jnp.dot(t, wfc_ref[...], preferred_element_type=jnp.float32) + bfc_ref[...])


def _tc3(m0, m1, d0, d1, b2, res, Wfc, bfc):
    g = _NP // _ROWS
    return pl.pallas_call(
        _tc3_body,
        grid=(g,),
        in_specs=[
            pl.BlockSpec((_ROWS, 64), lambda i: (i, 0)),
            pl.BlockSpec((_ROWS, 64), lambda i: (i, 0)),
            pl.BlockSpec((_ROWS, 1), lambda i: (i, 0)),
            pl.BlockSpec((_ROWS, 1), lambda i: (i, 0)),
            pl.BlockSpec((1, 64), lambda i: (0, 0)),
            pl.BlockSpec((_ROWS, 64), lambda i: (i, 0)),
            pl.BlockSpec((64, 1), lambda i: (0, 0)),
            pl.BlockSpec((1, 1), lambda i: (0, 0)),
        ],
        out_specs=pl.BlockSpec((_ROWS, 1), lambda i: (i, 0)),
        out_shape=jax.ShapeDtypeStruct((_NP, 1), jnp.float32),
    )(m0, m1, d0, d1, b2, res, Wfc, bfc)


def _make_sc_edge(F):
    mesh = plsc.VectorSubcoreMesh(core_axis_name="c", subcore_axis_name="s")

    @functools.partial(
        pl.kernel,
        out_type=[
            jax.ShapeDtypeStruct((_NCORE, _NP, F), jnp.bfloat16),
            jax.ShapeDtypeStruct((_NCORE, _DR, _L), jnp.float32),
        ],
        mesh=mesh,
        compiler_params=pltpu.CompilerParams(needs_layout_passes=False,
                                             use_tc_tiling_on_sc=False),
        scratch_types=[
            pltpu.VMEM((_CHM, _B), jnp.int32),
            pltpu.VMEM((_CHM, _B), jnp.int32),
            pltpu.VMEM((_NT,), jnp.float32),
            pltpu.VMEM((_NT,), jnp.float32),
            pltpu.VMEM((_B, F), jnp.bfloat16),
            pltpu.VMEM((_B, F), jnp.bfloat16),
            pltpu.VMEM((_B, F), jnp.bfloat16),
            pltpu.VMEM((_B, F), jnp.bfloat16),
            pltpu.VMEM((_B,), jnp.float32),
            pltpu.VMEM((_DR, _L), jnp.float32),
            pltpu.VMEM((_DB, _MB), jnp.int32),
            pltpu.VMEM_SHARED((_NP, F), jnp.bfloat16),
            pltpu.VMEM_SHARED((_DR, _L), jnp.float32),
            pltpu.SemaphoreType.DMA,
            pltpu.SemaphoreType.DMA,
            pltpu.SemaphoreType.DMA,
            pltpu.SemaphoreType.DMA,
        ],
    )
    def sc_edge(src3, dst3, as_t, ad_t, h, zrows, zden, ident, mout, dout,
                src_b, dst_b, as_b, ad_b, gbuf0, gbuf1, sbuf0, sbuf1, pbuf,
                dloc, ident_b, acc, dacc, sem_g0, sem_g1, sem_s0, sem_s1):
        c = lax.axis_index("c")
        s = lax.axis_index("s")
        base = jnp.where(c == 0, s * _CH0, _NSUB * _CH0 + s * _CH1)
        nth = jnp.where(c == 0, _CH0 // 2, _CH1 // 2)
        pltpu.sync_copy(src3.at[pl.ds(base, _CHM)], src_b)
        pltpu.sync_copy(dst3.at[pl.ds(base, _CHM)], dst_b)
        pltpu.sync_copy(as_t, as_b)
        pltpu.sync_copy(ad_t, ad_b)
        pltpu.sync_copy(ident, ident_b)
        pltpu.sync_copy(zden, dloc)
        pltpu.sync_copy(zrows, acc.at[pl.ds(s * _RPT, _RPT)])
        pltpu.sync_copy(zden.at[pl.ds(0, _DRT)], dacc.at[pl.ds(s * _DRT, _DRT)])
        plsc.subcore_barrier()

        def compute_p(j):
            for g in range(_B // _L):
                sv = src_b[j, pl.ds(g * _L, _L)]
                dv = dst_b[j, pl.ds(g * _L, _L)]
                u = plsc.load_gather(as_b, [sv]) + plsc.load_gather(ad_b, [dv])
                p = jnp.exp(jnp.maximum(u, 0.2 * u))
                pbuf[pl.ds(g * _L, _L)] = p
                plsc.addupdate_scatter(
                    dloc,
                    [lax.shift_right_logical(dv, 4), lax.bitwise_and(dv, 15)],
                    p)

        def scale(gb, sb):
            for g in range(_B // _L):
                pg = pbuf[pl.ds(g * _L, _L)]
                for k in range(_L):
                    e = g * _L + k
                    pk = lax.gather(
                        pg, jnp.full((_L, 1), k, jnp.int32),
                        lax.GatherDimensionNumbers(
                            offset_dims=(), collapsed_slice_dims=(0,),
                            start_index_map=(0,)),
                        slice_sizes=(1,),
                        mode=lax.GatherScatterMode.PROMISE_IN_BOUNDS)
                    pkb = plsc.pack(pk, pk, format=plsc.PackFormat.INTERLEAVED)
                    for q in range(F // (2 * _L)):
                        sb[e, pl.ds(q * 2 * _L, 2 * _L)] = (
                            gb[e, pl.ds(q * 2 * _L, 2 * _L)] * pkb)

        def body(t, carry):
            j0 = 2 * t
            j1 = j0 + 1
            cp0 = pltpu.async_copy(h.at[src_b.at[j0]], gbuf0, sem_g0)
            cp1 = pltpu.async_copy(h.at[src_b.at[j1]], gbuf1, sem_g1)
            compute_p(j0)
            cp0.wait()
            scale(gbuf0, sbuf0)
            cs0 = pltpu.async_copy(sbuf0, acc.at[dst_b.at[j0]], sem_s0, add=True)
            compute_p(j1)
            cp1.wait()
            scale(gbuf1, sbuf1)
            cs1 = pltpu.async_copy(sbuf1, acc.at[dst_b.at[j1]], sem_s1, add=True)
            cs0.wait()
            cs1.wait()
            return carry

        lax.fori_loop(0, nth, body, 0)
        for r in range(_DB):
            pltpu.sync_copy(dloc.at[pl.ds(r * _MB, _MB)],
                            dacc.at[ident_b.at[r]], add=True)
        plsc.subcore_barrier()
        pltpu.sync_copy(acc.at[pl.ds(s * _RPT, _RPT)],
                        mout.at[c, pl.ds(s * _RPT, _RPT)])
        pltpu.sync_copy(dacc.at[pl.ds(s * _DRT, _DRT)],
                        dout.at[c, pl.ds(s * _DRT, _DRT)])

    return sc_edge


_sc_edge_32 = _make_sc_edge(32)
_sc_edge_64 = _make_sc_edge(64)


def kernel(x, edge_index, W1, a1s, a1d, b1, W2, a2s, a2d, b2, Wres, bres, Wfc, bfc):
    f32 = jnp.float32
    xp = jnp.pad(x.astype(f32), ((0, _NP - _N), (0, 0)))

    loop = jnp.arange(_N, dtype=jnp.int32)
    npad = _EPAD - _E - _N
    nx = _XCH * _B
    padd = _N + jnp.arange(npad, dtype=jnp.int32) % (_NP - _N)
    src = jnp.concatenate(
        [edge_index[0].astype(jnp.int32), loop,
         jnp.zeros((npad + nx,), jnp.int32)]).reshape(_TOTCH + _XCH, _B)
    dst = jnp.concatenate(
        [edge_index[1].astype(jnp.int32), loop, padd,
         jnp.zeros((nx,), jnp.int32)]).reshape(_TOTCH + _XCH, _B)
    ident = jnp.arange(_DR, dtype=jnp.int32).reshape(_DB, _MB)

    z32 = jnp.zeros((_RPT, 32), jnp.bfloat16)
    z64 = jnp.zeros((_RPT, 64), jnp.bfloat16)
    zden = jnp.zeros((_DR, _L), f32)

    h1, as1, ad1, res = _tc1(xp, W1, a1s.reshape(32, 1), a1d.reshape(32, 1),
                             Wres, bres.reshape(1, 64))
    m1, d1 = _sc_edge_32(src, dst, as1[:_NT, 0], ad1[:_NT, 0], h1, z32, zden, ident)
    h2, as2, ad2 = _tc2(m1[0], m1[1],
                        d1[0].reshape(_NP, 1), d1[1].reshape(_NP, 1),
                        b1.reshape(1, 32), W2,
                        a2s.reshape(64, 1), a2d.reshape(64, 1))
    m2, d2 = _sc_edge_64(src, dst, as2[:_NT, 0], ad2[:_NT, 0], h2, z64, zden, ident)
    out = _tc3(m2[0], m2[1],
               d2[0].reshape(_NP, 1), d2[1].reshape(_NP, 1),
               b2.reshape(1, 64), res, Wfc, bfc.reshape(1, 1))
    return out[:_N]


# speedup vs baseline: 2.3026x; 1.0823x over previous
import functools

import jax
import jax.numpy as jnp
from jax import lax
from jax.experimental import pallas as pl
from jax.experimental.pallas import tpu as pltpu
from jax.experimental.pallas import tpu_sc as plsc

_N = 10000
_D = 128
_E = 320000

_L = 16
_NSUB = 16
_NCORE = 2
_NW = _NCORE * _NSUB
_B = 256
_CH0 = 58
_CH1 = 24
_CHM = max(_CH0, _CH1)
_TOTCH = _NSUB * (_CH0 + _CH1)
_EPAD = _TOTCH * _B
_XCH = _CHM - min(_CH0, _CH1)
_NP = 10240
_NT = 10016
_RPT = _NP // _NSUB
_DR = _NP // _L
_MB = 128
_DB = _DR // _MB
_DRT = _DR // _NSUB

_ROWS = 1024


def _tc1_body(x_ref, w1_ref, a1s_ref, a1d_ref, wres_ref, bres_ref,
              h1_ref, as_ref, ad_ref, res_ref):
    xb = x_ref[...]
    h = jnp.dot(xb, w1_ref[...], preferred_element_type=jnp.float32)
    h1_ref[...] = h.astype(jnp.bfloat16)
    as_ref[...] = jnp.dot(h, a1s_ref[...], preferred_element_type=jnp.float32)
    ad_ref[...] = jnp.dot(h, a1d_ref[...], preferred_element_type=jnp.float32)
    res_ref[...] = (jnp.dot(xb, wres_ref[...], preferred_element_type=jnp.float32)
                    + bres_ref[...])


def _tc1(xp, W1, a1s, a1d, Wres, bres):
    g = _NP // _ROWS
    return pl.pallas_call(
        _tc1_body,
        grid=(g,),
        in_specs=[
            pl.BlockSpec((_ROWS, _D), lambda i: (i, 0)),
            pl.BlockSpec((_D, 32), lambda i: (0, 0)),
            pl.BlockSpec((32, 1), lambda i: (0, 0)),
            pl.BlockSpec((32, 1), lambda i: (0, 0)),
            pl.BlockSpec((_D, 64), lambda i: (0, 0)),
            pl.BlockSpec((1, 64), lambda i: (0, 0)),
        ],
        out_specs=[
            pl.BlockSpec((_ROWS, 32), lambda i: (i, 0)),
            pl.BlockSpec((_ROWS, 1), lambda i: (i, 0)),
            pl.BlockSpec((_ROWS, 1), lambda i: (i, 0)),
            pl.BlockSpec((_ROWS, 64), lambda i: (i, 0)),
        ],
        out_shape=[
            jax.ShapeDtypeStruct((_NP, 32), jnp.bfloat16),
            jax.ShapeDtypeStruct((_NP, 1), jnp.float32),
            jax.ShapeDtypeStruct((_NP, 1), jnp.float32),
            jax.ShapeDtypeStruct((_NP, 64), jnp.float32),
        ],
    )(xp, W1, a1s, a1d, Wres, bres)


def _tc2_body(m0_ref, m1_ref, d0_ref, d1_ref, b1_ref, w2_ref, a2s_ref, a2d_ref,
              h2_ref, as_ref, ad_ref):
    num = (m0_ref[...].astype(jnp.float32) + m1_ref[...].astype(jnp.float32))
    den = d0_ref[...] + d1_ref[...] + 1e-16
    z = jnp.maximum(num / den + b1_ref[...], 0.0)
    h2 = jnp.dot(z, w2_ref[...], preferred_element_type=jnp.float32)
    h2_ref[...] = h2.astype(jnp.bfloat16)
    as_ref[...] = jnp.dot(h2, a2s_ref[...], preferred_element_type=jnp.float32)
    ad_ref[...] = jnp.dot(h2, a2d_ref[...], preferred_element_type=jnp.float32)


def _tc2(m0, m1, d0, d1, b1, W2, a2s, a2d):
    g = _NP // _ROWS
    return pl.pallas_call(
        _tc2_body,
        grid=(g,),
        in_specs=[
            pl.BlockSpec((_ROWS, 32), lambda i: (i, 0)),
            pl.BlockSpec((_ROWS, 32), lambda i: (i, 0)),
            pl.BlockSpec((_ROWS, 1), lambda i: (i, 0)),
            pl.BlockSpec((_ROWS, 1), lambda i: (i, 0)),
            pl.BlockSpec((1, 32), lambda i: (0, 0)),
            pl.BlockSpec((32, 64), lambda i: (0, 0)),
            pl.BlockSpec((64, 1), lambda i: (0, 0)),
            pl.BlockSpec((64, 1), lambda i: (0, 0)),
        ],
        out_specs=[
            pl.BlockSpec((_ROWS, 64), lambda i: (i, 0)),
            pl.BlockSpec((_ROWS, 1), lambda i: (i, 0)),
            pl.BlockSpec((_ROWS, 1), lambda i: (i, 0)),
        ],
        out_shape=[
            jax.ShapeDtypeStruct((_NP, 64), jnp.bfloat16),
            jax.ShapeDtypeStruct((_NP, 1), jnp.float32),
            jax.ShapeDtypeStruct((_NP, 1), jnp.float32),
        ],
    )(m0, m1, d0, d1, b1, W2, a2s, a2d)


def _tc3_body(m0_ref, m1_ref, d0_ref, d1_ref, b2_ref, res_ref, wfc_ref, bfc_ref,
              o_ref):
    num = (m0_ref[...].astype(jnp.float32) + m1_ref[...].astype(jnp.float32))
    den = d0_ref[...] + d1_ref[...] + 1e-16
    z = jnp.maximum(num / den + b2_ref[...], 0.0)
    t = z + res_ref[...]
    o_ref[...] = jax.nn.sigmoid(
        jnp.dot(t, wfc_ref[...], preferred_element_type=jnp.float32) + bfc_ref[...])


def _tc3(m0, m1, d0, d1, b2, res, Wfc, bfc):
    g = _NP // _ROWS
    return pl.pallas_call(
        _tc3_body,
        grid=(g,),
        in_specs=[
            pl.BlockSpec((_ROWS, 64), lambda i: (i, 0)),
            pl.BlockSpec((_ROWS, 64), lambda i: (i, 0)),
            pl.BlockSpec((_ROWS, 1), lambda i: (i, 0)),
            pl.BlockSpec((_ROWS, 1), lambda i: (i, 0)),
            pl.BlockSpec((1, 64), lambda i: (0, 0)),
            pl.BlockSpec((_ROWS, 64), lambda i: (i, 0)),
            pl.BlockSpec((64, 1), lambda i: (0, 0)),
            pl.BlockSpec((1, 1), lambda i: (0, 0)),
        ],
        out_specs=pl.BlockSpec((_ROWS, 1), lambda i: (i, 0)),
        out_shape=jax.ShapeDtypeStruct((_NP, 1), jnp.float32),
    )(m0, m1, d0, d1, b2, res, Wfc, bfc)


def _make_sc_edge(F):
    mesh = plsc.VectorSubcoreMesh(core_axis_name="c", subcore_axis_name="s")

    @functools.partial(
        pl.kernel,
        out_type=[
            jax.ShapeDtypeStruct((_NCORE, _NP, F), jnp.bfloat16),
            jax.ShapeDtypeStruct((_NCORE, _DR, _L), jnp.float32),
        ],
        mesh=mesh,
        compiler_params=pltpu.CompilerParams(needs_layout_passes=False,
                                             use_tc_tiling_on_sc=False),
        scratch_types=[
            pltpu.VMEM((_CHM, _B), jnp.int32),
            pltpu.VMEM((_CHM, _B), jnp.int32),
            pltpu.VMEM((_NT,), jnp.float32),
            pltpu.VMEM((_NT,), jnp.float32),
            pltpu.VMEM((_B, F), jnp.bfloat16),
            pltpu.VMEM((_B, F), jnp.bfloat16),
            pltpu.VMEM((_B, F), jnp.bfloat16),
            pltpu.VMEM((_B, F), jnp.bfloat16),
            pltpu.VMEM((_B,), jnp.float32),
            pltpu.VMEM((_DR, _L), jnp.float32),
            pltpu.VMEM((_DB, _MB), jnp.int32),
            pltpu.VMEM_SHARED((_NP, F), jnp.bfloat16),
            pltpu.VMEM_SHARED((_DR, _L), jnp.float32),
            pltpu.SemaphoreType.DMA,
            pltpu.SemaphoreType.DMA,
            pltpu.SemaphoreType.DMA,
            pltpu.SemaphoreType.DMA,
        ],
    )
    def sc_edge(src3, dst3, as_t, ad_t, h, zrows, zden, ident, mout, dout,
                src_b, dst_b, as_b, ad_b, gbuf0, gbuf1, sbuf0, sbuf1, pbuf,
                dloc, ident_b, acc, dacc, sem_g0, sem_g1, sem_s0, sem_s1):
        c = lax.axis_index("c")
        s = lax.axis_index("s")
        base = jnp.where(c == 0, s * _CH0, _NSUB * _CH0 + s * _CH1)
        nth = jnp.where(c == 0, _CH0 // 2, _CH1 // 2)
        pltpu.sync_copy(src3.at[pl.ds(base, _CHM)], src_b)
        pltpu.sync_copy(dst3.at[pl.ds(base, _CHM)], dst_b)
        pltpu.sync_copy(as_t, as_b)
        pltpu.sync_copy(ad_t, ad_b)
        pltpu.sync_copy(ident, ident_b)
        pltpu.sync_copy(zden, dloc)
        pltpu.sync_copy(zrows, acc.at[pl.ds(s * _RPT, _RPT)])
        pltpu.sync_copy(zden.at[pl.ds(0, _DRT)], dacc.at[pl.ds(s * _DRT, _DRT)])
        plsc.subcore_barrier()

        def compute_p(j):
            for g in range(_B // _L):
                sv = src_b[j, pl.ds(g * _L, _L)]
                dv = dst_b[j, pl.ds(g * _L, _L)]
                u = plsc.load_gather(as_b, [sv]) + plsc.load_gather(ad_b, [dv])
                p = jnp.exp(jnp.maximum(u, 0.2 * u))
                pbuf[pl.ds(g * _L, _L)] = p
                plsc.addupdate_scatter(
                    dloc,
                    [lax.shift_right_logical(dv, 4), lax.bitwise_and(dv, 15)],
                    p)

        def scale(gb, sb):
            for g in range(_B // _L):
                pg = pbuf[pl.ds(g * _L, _L)]
                for k in range(_L):
                    e = g * _L + k
                    pk = lax.gather(
                        pg, jnp.full((_L, 1), k, jnp.int32),
                        lax.GatherDimensionNumbers(
                            offset_dims=(), collapsed_slice_dims=(0,),
                            start_index_map=(0,)),
                        slice_sizes=(1,),
                        mode=lax.GatherScatterMode.PROMISE_IN_BOUNDS)
                    pkb = plsc.pack(pk, pk, format=plsc.PackFormat.INTERLEAVED)
                    for q in range(F // (2 * _L)):
                        sb[e, pl.ds(q * 2 * _L, 2 * _L)] = (
                            gb[e, pl.ds(q * 2 * _L, 2 * _L)] * pkb)

        pltpu.async_copy(h.at[src_b.at[0]], gbuf0, sem_g0)
        pltpu.async_copy(h.at[src_b.at[1]], gbuf1, sem_g1)

        def body(t, carry):
            j0 = 2 * t
            j1 = j0 + 1
            compute_p(j0)
            pltpu.make_async_copy(h.at[src_b.at[j0]], gbuf0, sem_g0).wait()

            @pl.when(t > 0)
            def _():
                pltpu.make_async_copy(sbuf0, acc.at[dst_b.at[j0 - 2]],
                                      sem_s0).wait()

            scale(gbuf0, sbuf0)
            pltpu.async_copy(sbuf0, acc.at[dst_b.at[j0]], sem_s0, add=True)

            @pl.when(t + 1 < nth)
            def _():
                pltpu.async_copy(h.at[src_b.at[j0 + 2]], gbuf0, sem_g0)

            compute_p(j1)
            pltpu.make_async_copy(h.at[src_b.at[j1]], gbuf1, sem_g1).wait()

            @pl.when(t > 0)
            def _():
                pltpu.make_async_copy(sbuf1, acc.at[dst_b.at[j1 - 2]],
                                      sem_s1).wait()

            scale(gbuf1, sbuf1)
            pltpu.async_copy(sbuf1, acc.at[dst_b.at[j1]], sem_s1, add=True)

            @pl.when(t + 1 < nth)
            def _():
                pltpu.async_copy(h.at[src_b.at[j1 + 2]], gbuf1, sem_g1)

            return carry

        lax.fori_loop(0, nth, body, 0)
        pltpu.make_async_copy(sbuf0, acc.at[dst_b.at[2 * nth - 2]],
                              sem_s0).wait()
        pltpu.make_async_copy(sbuf1, acc.at[dst_b.at[2 * nth - 1]],
                              sem_s1).wait()
        for r in range(_DB):
            pltpu.sync_copy(dloc.at[pl.ds(r * _MB, _MB)],
                            dacc.at[ident_b.at[r]], add=True)
        plsc.subcore_barrier()
        pltpu.sync_copy(acc.at[pl.ds(s * _RPT, _RPT)],
                        mout.at[c, pl.ds(s * _RPT, _RPT)])
        pltpu.sync_copy(dacc.at[pl.ds(s * _DRT, _DRT)],
                        dout.at[c, pl.ds(s * _DRT, _DRT)])

    return sc_edge


_sc_edge_32 = _make_sc_edge(32)
_sc_edge_64 = _make_sc_edge(64)


def kernel(x, edge_index, W1, a1s, a1d, b1, W2, a2s, a2d, b2, Wres, bres, Wfc, bfc):
    f32 = jnp.float32
    xp = jnp.pad(x.astype(f32), ((0, _NP - _N), (0, 0)))

    loop = jnp.arange(_N, dtype=jnp.int32)
    npad = _EPAD - _E - _N
    nx = _XCH * _B
    padd = _N + jnp.arange(npad, dtype=jnp.int32) % (_NP - _N)
    src = jnp.concatenate(
        [edge_index[0].astype(jnp.int32), loop,
         jnp.zeros((npad + nx,), jnp.int32)]).reshape(_TOTCH + _XCH, _B)
    dst = jnp.concatenate(
        [edge_index[1].astype(jnp.int32), loop, padd,
         jnp.zeros((nx,), jnp.int32)]).reshape(_TOTCH + _XCH, _B)
    ident = jnp.arange(_DR, dtype=jnp.int32).reshape(_DB, _MB)

    z32 = jnp.zeros((_RPT, 32), jnp.bfloat16)
    z64 = jnp.zeros((_RPT, 64), jnp.bfloat16)
    zden = jnp.zeros((_DR, _L), f32)

    h1, as1, ad1, res = _tc1(xp, W1, a1s.reshape(32, 1), a1d.reshape(32, 1),
                             Wres, bres.reshape(1, 64))
    m1, d1 = _sc_edge_32(src, dst, as1[:_NT, 0], ad1[:_NT, 0], h1, z32, zden, ident)
    h2, as2, ad2 = _tc2(m1[0], m1[1],
                        d1[0].reshape(_NP, 1), d1[1].reshape(_NP, 1),
                        b1.reshape(1, 32), W2,
                        a2s.reshape(64, 1), a2d.reshape(64, 1))
    m2, d2 = _sc_edge_64(src, dst, as2[:_NT, 0], ad2[:_NT, 0], h2, z64, zden, ident)
    out = _tc3(m2[0], m2[1],
               d2[0].reshape(_NP, 1), d2[1].reshape(_NP, 1),
               b2.reshape(1, 64), res, Wfc, bfc.reshape(1, 1))
    return out[:_N]
